# baseline (speedup 1.0000x reference)
import functools

import jax
import jax.numpy as jnp
from jax.experimental import pallas as pl
from jax.experimental.pallas import tpu as pltpu

LANE = 128
BN_EPS = 1e-5


def _round_up(a, m):
    return ((a + m - 1) // m) * m


def _block_sums(x_ref, hw):
    n_full = hw // LANE
    tail = hw % LANE
    acc = jnp.zeros(x_ref.shape[:2] + (LANE,), jnp.float32)
    for j in range(n_full):
        acc = acc + x_ref[:, :, j * LANE:(j + 1) * LANE].astype(jnp.float32)
    if tail:
        lane = jax.lax.broadcasted_iota(jnp.int32, (1, 1, LANE), 2)
        chunk = x_ref[:, :, n_full * LANE:(n_full + 1) * LANE]
        acc = acc + jnp.where(lane < tail, chunk.astype(jnp.float32), 0.0)
    return jnp.sum(acc, axis=2)


def _fused_kernel(x_ref, wproj_h, gamma_h, beta_h, wcls_h,
                  gfeat_h, feat_h, cls_h,
                  psum, wp_v, ga_v, be_v, wc_v, gf_v, ft_v, cl_v, sem,
                  *, nblocks, bblk, hw, inv_hw):
    k = pl.program_id(0)

    @pl.when(k == 0)
    def _fetch_weights():
        pltpu.make_async_copy(wproj_h, wp_v, sem.at[0]).start()
        pltpu.make_async_copy(gamma_h, ga_v, sem.at[1]).start()
        pltpu.make_async_copy(beta_h, be_v, sem.at[2]).start()
        pltpu.make_async_copy(wcls_h, wc_v, sem.at[3]).start()

    psum[pl.ds(k * bblk, bblk)] = _block_sums(x_ref, hw)

    @pl.when(k == nblocks - 1)
    def _head():
        pltpu.make_async_copy(wproj_h, wp_v, sem.at[0]).wait()
        pltpu.make_async_copy(gamma_h, ga_v, sem.at[1]).wait()
        pltpu.make_async_copy(beta_h, be_v, sem.at[2]).wait()
        pltpu.make_async_copy(wcls_h, wc_v, sem.at[3]).wait()
        pooled = psum[...] * inv_hw
        gfeat = jnp.dot(pooled, wp_v[...],
                        preferred_element_type=jnp.float32)
        gf_v[...] = gfeat
        mu = jnp.mean(gfeat, axis=0, keepdims=True)
        d = gfeat - mu
        var = jnp.mean(d * d, axis=0, keepdims=True)
        feat = d * jax.lax.rsqrt(var + BN_EPS) * ga_v[...] + be_v[...]
        ft_v[...] = feat
        cl_v[...] = jax.lax.dot_general(
            feat, wc_v[...], (((1,), (1,)), ((), ())),
            preferred_element_type=jnp.float32)
        cp_g = pltpu.make_async_copy(gf_v, gfeat_h, sem.at[4])
        cp_f = pltpu.make_async_copy(ft_v, feat_h, sem.at[5])
        cp_c = pltpu.make_async_copy(cl_v, cls_h, sem.at[6])
        cp_g.start()
        cp_f.start()
        cp_c.start()
        cp_g.wait()
        cp_f.wait()
        cp_c.wait()


def kernel(x, wproj, gamma, beta, wcls):
    B, C, H, W = x.shape
    HW = H * W
    P = wproj.shape[1]
    NC = wcls.shape[0]
    hwpad = _round_up(HW, LANE)

    row_bytes = C * hwpad * jnp.dtype(x.dtype).itemsize
    bblk = 1
    for cand in (8, 4, 2):
        if B % cand == 0 and 2 * cand * row_bytes <= 36 * 1024 * 1024:
            bblk = cand
            break
    nblocks = B // bblk

    vmem_limit = int(min(56 * 1024 * 1024,
                         2 * bblk * row_bytes + 6 * 1024 * 1024))

    x3 = x.reshape(B, C, HW)
    hbm = pl.BlockSpec(memory_space=pltpu.MemorySpace.HBM)
    gfeat, feat, cls_score = pl.pallas_call(
        functools.partial(_fused_kernel, nblocks=nblocks, bblk=bblk,
                          hw=HW, inv_hw=1.0 / float(HW)),
        out_shape=(
            jax.ShapeDtypeStruct((B, P), jnp.float32),
            jax.ShapeDtypeStruct((B, P), jnp.float32),
            jax.ShapeDtypeStruct((B, NC), jnp.float32),
        ),
        grid=(nblocks,),
        in_specs=[pl.BlockSpec((bblk, C, hwpad), lambda k: (k, 0, 0)),
                  hbm, hbm, hbm, hbm],
        out_specs=(hbm, hbm, hbm),
        scratch_shapes=[
            pltpu.VMEM((B, C), jnp.float32),
            pltpu.VMEM((C, P), jnp.float32),
            pltpu.VMEM((1, P), jnp.float32),
            pltpu.VMEM((1, P), jnp.float32),
            pltpu.VMEM((NC, P), jnp.float32),
            pltpu.VMEM((B, P), jnp.float32),
            pltpu.VMEM((B, P), jnp.float32),
            pltpu.VMEM((B, NC), jnp.float32),
            pltpu.SemaphoreType.DMA((7,)),
        ],
        compiler_params=pltpu.CompilerParams(
            dimension_semantics=("arbitrary",),
            vmem_limit_bytes=vmem_limit,
        ),
    )(x3, wproj.astype(jnp.float32), gamma.reshape(1, P).astype(jnp.float32),
      beta.reshape(1, P).astype(jnp.float32), wcls.astype(jnp.float32))

    return cls_score, gfeat, feat
